# baseline (speedup 1.0000x reference)
import functools

import jax
import jax.numpy as jnp
from jax import lax
from jax.experimental import pallas as pl
from jax.experimental.pallas import tpu as pltpu
from jax.experimental.pallas import tpu_sc as plsc

B, L = 4096, 200
EMB = 14
HID = 16
N = B * L
VOC = 1000000
W_CLAMP = (VOC * HID - EMB) // EMB

_info = plsc.get_sparse_core_info()
NC, NS, LANES = _info.num_cores, _info.num_subcores, _info.num_lanes
NW = NC * NS
BPW = B // NW
BB = 4
BLK = BB * L
NBLK = BPW // BB
PER_W = BPW * L
NWIN = 2 * BLK
_SEGS = [(s, min(128, NWIN - s)) for s in range(0, NWIN, 128)]

_mesh = plsc.VectorSubcoreMesh(core_axis_name="c", subcore_axis_name="s")


@functools.partial(
    pl.kernel,
    mesh=_mesh,
    out_type=jax.ShapeDtypeStruct((N * HID,), jnp.float32),
    scratch_types=[
        pltpu.VMEM((2, BB, L, 2), jnp.float32),
        pltpu.VMEM((2 * BLK,), jnp.int32),
        pltpu.VMEM((2 * BLK,), jnp.int32),
        pltpu.VMEM((2 * BLK,), jnp.float32),
        pltpu.VMEM((2 * NWIN,), jnp.int32),
        pltpu.VMEM((2 * NWIN, EMB), jnp.float32),
        pltpu.VMEM((2 * BLK * HID,), jnp.float32),
        pltpu.SemaphoreType.DMA,
        pltpu.SemaphoreType.DMA,
    ],
    compiler_params=pltpu.CompilerParams(
        needs_layout_passes=False,
        use_tc_tiling_on_sc=False,
    ),
)
def _sc_embed(table_h, x_h, out_h, x_v, w1_v, o_v, dur_v, wl_v, win_v, out_v,
              sem_g, sem_o):
    wid = lax.axis_index("s") * NC + lax.axis_index("c")
    w_base = wid * PER_W
    lane = lax.iota(jnp.int32, LANES)
    c0 = jnp.zeros((LANES,), jnp.int32)
    c1 = jnp.ones((LANES,), jnp.int32)
    c_emb = jnp.minimum(lane, EMB - 1)
    m_emb = lane < EMB

    def stage_extract(g, s):
        pltpu.sync_copy(x_h.at[pl.ds(wid * BPW + g * BB, BB)], x_v.at[s])
        s_b = jnp.full((LANES,), s, jnp.int32)

        def extract_body(j, c):
            r = j * LANES + lane
            bvec = r // L
            lvec = r - bvec * L
            fidx = plsc.load_gather(x_v, [s_b, bvec, lvec, c0])
            fdur = plsc.load_gather(x_v, [s_b, bvec, lvec, c1])
            ii = fidx.astype(jnp.int32)
            t = (ii * 8) // 7
            o = ii * 16 - t * 14
            w1_v[pl.ds(s * BLK + j * LANES, LANES)] = t
            o_v[pl.ds(s * BLK + j * LANES, LANES)] = o
            dur_v[pl.ds(s * BLK + j * LANES, LANES)] = fdur
            return c

        lax.fori_loop(0, BLK // LANES, extract_body, 0)

        def winlist_body(j, c):
            k = j * LANES + lane
            t = plsc.load_gather(w1_v, [s * BLK + (k >> 1)])
            wl_v[pl.ds(s * NWIN + j * LANES, LANES)] = jnp.minimum(
                t + (k & 1), W_CLAMP
            )
            return c

        lax.fori_loop(0, NWIN // LANES, winlist_body, 0)

    def fire_gathers(s):
        for off, ln in _SEGS:
            pltpu.async_copy(
                table_h.at[wl_v.at[pl.ds(s * NWIN + off, ln)]],
                win_v.at[pl.ds(s * NWIN + off, ln)],
                sem_g,
            )

    def drain_gathers(s):
        for off, ln in _SEGS:
            pltpu.make_async_copy(
                table_h.at[wl_v.at[pl.ds(s * NWIN + off, ln)]],
                win_v.at[pl.ds(s * NWIN + off, ln)],
                sem_g,
            ).wait()

    def out_desc(g, s):
        base = (w_base + g * BLK) * HID
        return pltpu.make_async_copy(
            out_v.at[pl.ds(s * BLK * HID, BLK * HID)],
            out_h.at[pl.ds(base, BLK * HID)],
            sem_o,
        )

    stage_extract(0, 0)
    fire_gathers(0)

    def block_body(g, carry):
        s = lax.rem(g, 2)
        s1 = 1 - s

        @pl.when(g + 1 < NBLK)
        def _():
            stage_extract(g + 1, s1)

            @pl.when(g >= 1)
            def _():
                out_desc(g - 1, s1).wait()

            fire_gathers(s1)

        drain_gathers(s)

        def asm_body(r, c):
            r_b = jnp.full((LANES,), r, jnp.int32)
            o_b = plsc.load_gather(o_v, [s * BLK + r_b])
            dur = plsc.load_gather(dur_v, [s * BLK + r_b])
            p = o_b + c_emb
            hi = (p >= EMB).astype(jnp.int32)
            seg = (2 * r_b + hi) >> 7
            phys = (s * NWIN * HID + 28 * r) + p + (seg << 8)
            emb = plsc.load_gather(win_v, [phys >> 4, phys & 15])
            out_v[pl.ds((s * BLK + r) * HID, HID)] = jnp.where(m_emb, emb, dur)
            return c

        lax.fori_loop(0, BLK, asm_body, 0)

        out_desc(g, s).start()
        return carry

    lax.fori_loop(0, NBLK, block_body, 0)

    out_desc(NBLK - 2, lax.rem(NBLK - 2, 2)).wait()
    out_desc(NBLK - 1, lax.rem(NBLK - 1, 2)).wait()


def kernel(x, table):
    out = _sc_embed(table, x)
    return out.reshape(B, L, HID)


# speedup vs baseline: 2.2354x; 2.2354x over previous
import functools

import jax
import jax.numpy as jnp
from jax import lax
from jax.experimental import pallas as pl
from jax.experimental.pallas import tpu as pltpu
from jax.experimental.pallas import tpu_sc as plsc

B, L = 4096, 200
EMB = 14
HID = 16
N = B * L

_info = plsc.get_sparse_core_info()
NC, NS, LANES = _info.num_cores, _info.num_subcores, _info.num_lanes
NW = NC * NS
BPW = B // NW
BB = 4
BLK = BB * L
NBLK = BPW // BB
PER_W = BPW * L
_SEGS = [(s, min(128, BLK - s)) for s in range(0, BLK, 128)]

_mesh = plsc.VectorSubcoreMesh(core_axis_name="c", subcore_axis_name="s")


@functools.partial(
    pl.kernel,
    mesh=_mesh,
    out_type=jax.ShapeDtypeStruct((B, L, HID), jnp.float32),
    scratch_types=[
        pltpu.VMEM((2 * BLK,), jnp.int32),
        pltpu.VMEM((2 * BLK,), jnp.float32),
        pltpu.VMEM((2 * BLK, HID), jnp.float32),
        pltpu.SemaphoreType.DMA,
        pltpu.SemaphoreType.DMA,
    ],
    compiler_params=pltpu.CompilerParams(
        needs_layout_passes=False,
        use_tc_tiling_on_sc=False,
    ),
)
def _sc_embed(table_h, idx_h, dur_h, out_h, idx_v, dur_v, out_v, sem_g, sem_o):
    wid = lax.axis_index("s") * NC + lax.axis_index("c")
    lane = lax.iota(jnp.int32, LANES)
    rr_off = lane >> 1
    c_fix = (lane & 1) + EMB

    def stage(g, s):
        base = wid * PER_W + g * BLK
        pltpu.sync_copy(idx_h.at[pl.ds(base, BLK)], idx_v.at[pl.ds(s * BLK, BLK)])
        pltpu.sync_copy(dur_h.at[pl.ds(base, BLK)], dur_v.at[pl.ds(s * BLK, BLK)])

    def fire_gathers(s):
        for off, ln in _SEGS:
            pltpu.async_copy(
                table_h.at[idx_v.at[pl.ds(s * BLK + off, ln)]],
                out_v.at[pl.ds(s * BLK + off, ln)],
                sem_g,
            )

    def drain_gathers(s):
        for off, ln in _SEGS:
            pltpu.make_async_copy(
                table_h.at[idx_v.at[pl.ds(s * BLK + off, ln)]],
                out_v.at[pl.ds(s * BLK + off, ln)],
                sem_g,
            ).wait()

    def out_descs(g, s):
        bbase = wid * BPW + g * BB
        return [
            pltpu.make_async_copy(
                out_v.at[pl.ds(s * BLK + k * L, L)],
                out_h.at[bbase + k],
                sem_o,
            )
            for k in range(BB)
        ]

    stage(0, 0)
    fire_gathers(0)

    def block_body(g, carry):
        s = lax.rem(g, 2)
        s1 = 1 - s

        @pl.when(g + 1 < NBLK)
        def _():
            stage(g + 1, s1)

            @pl.when(g >= 1)
            def _():
                for d in out_descs(g - 1, s1):
                    d.wait()

            fire_gathers(s1)

        drain_gathers(s)

        def fix_body(j, c):
            r_idx = s * BLK + j * 8 + rr_off
            val = plsc.load_gather(dur_v, [r_idx])
            plsc.store_scatter(out_v, [r_idx, c_fix], val)
            return c

        lax.fori_loop(0, BLK // 8, fix_body, 0)

        for d in out_descs(g, s):
            d.start()
        return carry

    lax.fori_loop(0, NBLK, block_body, 0)

    for d in out_descs(NBLK - 2, lax.rem(NBLK - 2, 2)):
        d.wait()
    for d in out_descs(NBLK - 1, lax.rem(NBLK - 1, 2)):
        d.wait()


def kernel(x, table):
    table16 = jnp.pad(table, ((0, 0), (0, HID - EMB)))
    idx = x[..., 0].astype(jnp.int32).reshape(N)
    dur = x[..., 1].reshape(N)
    return _sc_embed(table16, idx, dur)
